# Initial kernel scaffold; baseline (speedup 1.0000x reference)
import jax
import jax.numpy as jnp
from jax.experimental import pallas as pl


def kernel(video_clips):
    raise NotImplementedError("write your pallas kernel here")


# speedup vs baseline: 7.9456x; 7.9456x over previous
import numpy as np
import jax
import jax.numpy as jnp
from jax.experimental import pallas as pl
from jax.experimental.pallas import tpu as pltpu

EPS = 1e-8
B, C, T, H, W = 16, 3, 32, 112, 112
NPIX = H * W
TOPK = int(0.1 * H * W)
LANES = 128
ONE_BITS = 0x3F800000


def _gauss1d_np(ks, sigma):
    x = np.arange(ks, dtype=np.float32) - (ks - 1) / 2.0
    g = np.exp(-0.5 * (x / sigma) ** 2).astype(np.float32)
    return g / g.sum()


def _blur_matrix():
    k1 = _gauss1d_np(15, 5.0).astype(np.float64)
    t = np.arange(-7, H + 7)
    r = np.where(t < 0, -t, np.where(t > H - 1, 2 * (H - 1) - t, t))
    M = np.zeros((H, H), np.float64)
    for a in range(15):
        for i in range(H):
            M[i, r[i + a]] += k1[a]
    return M.astype(np.float32)


def _window_np():
    ky = _gauss1d_np(H, H / 3.0)
    kx = _gauss1d_np(W, W / 3.0)
    k = np.outer(ky, kx)
    return (k / k.max()).astype(np.float32)


_BLUR_M = _blur_matrix()
_WIN = _window_np()


def _reflect_pad_matrix():
    t = np.arange(-7, H + 7)
    r = np.where(t < 0, -t, np.where(t > H - 1, 2 * (H - 1) - t, t))
    P = np.zeros((H + 14, H), np.float32)
    P[np.arange(H + 14), r] = 1.0
    return P


def _band_placement():
    E = np.zeros((15, H, H + 14), np.float32)
    for a in range(15):
        E[a, np.arange(H), np.arange(H) + a] = 1.0
    return E


_PAD_P = _reflect_pad_matrix()
_BAND_E = _band_placement()


def _stats_kernel(v_ref, diff_ref, mean_ref):
    v = v_ref[0]
    acc = None
    for c in range(C):
        xc = v[c * T:(c + 1) * T]
        d = jnp.abs(xc[:-1] - xc[1:])
        acc = d if acc is None else acc + d
    diff_ref[0] = acc.mean(axis=0)
    for c in range(C):
        mean_ref[0, c] = v[c * T + 1:(c + 1) * T].mean(axis=0)


def _dot(a, b):
    return jnp.dot(a, b, preferred_element_type=jnp.float32,
                   precision=jax.lax.Precision.HIGHEST)


def _mid_kernel(diff_ref, rgb_ref, A_ref, P_ref, M_ref, win_ref, out_ref):
    Mb = M_ref[...]

    xbf = diff_ref[0].astype(jnp.bfloat16).astype(jnp.float32)
    P = P_ref[...]
    xp = _dot(_dot(P, xbf), P.T)
    acc = jnp.zeros((H, W), jnp.float32)
    for b in range(15):
        Tb = _dot(A_ref[b], xp)
        acc = acc + Tb[:, b:b + W]
    y = acc
    y = y - jnp.min(y)
    y = y / (jnp.max(y) + EPS)
    mask = y * win_ref[...]

    r = rgb_ref[0, 0]
    g = rgb_ref[0, 1]
    bl = rgb_ref[0, 2]
    maxc = jnp.maximum(jnp.maximum(r, g), bl)
    minc = jnp.minimum(jnp.minimum(r, g), bl)
    vch = maxc
    deltac = maxc - minc
    s = deltac / (maxc + EPS)
    dsafe = jnp.where(deltac == 0.0, 1.0, deltac)
    rc = (maxc - r) / dsafe
    gc = (maxc - g) / dsafe
    bc = (maxc - bl) / dsafe
    h = jnp.where(maxc == r, bc - gc,
                  jnp.where(maxc == g, 2.0 + rc - bc, 4.0 + gc - rc))
    h = (h / 6.0) % 1.0
    h = jnp.where(deltac == 0.0, 0.0, h)

    hx = (s * jnp.cos(h * (2 * np.pi)) + 1.0) / 2.0
    hy = (s * jnp.sin(h * (2 * np.pi)) + 1.0) / 2.0
    hq = jnp.round(hx * 4.0 + 1.0)
    sq = jnp.round(hy * 4.0 + 1.0)
    vq = jnp.round(vch * 4.0 + 1.0)
    cmap = hq + (sq - 1.0) * 5.0 + (vq - 1.0) * 25.0

    mbits = jax.lax.bitcast_convert_type(mask, jnp.int32)
    idx = (jax.lax.broadcasted_iota(jnp.int32, (H, W), 0) * W
           + jax.lax.broadcasted_iota(jnp.int32, (H, W), 1))

    def fg_body(_, carry):
        lo, hi = carry
        mid = lo + (hi - lo) // 2
        ok = jnp.sum((mbits >= mid).astype(jnp.int32)) >= TOPK
        return jnp.where(ok, mid, lo), jnp.where(ok, hi, mid)

    tfg, _ = jax.lax.fori_loop(
        0, 31, fg_body, (jnp.int32(0), jnp.int32(ONE_BITS + 1)))

    def bg_body(_, carry):
        lo, hi = carry
        mid = (lo + hi) // 2
        ok = jnp.sum((mbits <= mid).astype(jnp.int32)) >= TOPK
        return jnp.where(ok, lo, mid + 1), jnp.where(ok, mid, hi)

    _, tbg = jax.lax.fori_loop(
        0, 31, bg_body, (jnp.int32(0), jnp.int32(ONE_BITS)))

    m_fg = TOPK - jnp.sum((mbits > tfg).astype(jnp.int32))
    tie_fg = mbits == tfg
    m_bg = TOPK - jnp.sum((mbits < tbg).astype(jnp.int32))
    tie_bg = mbits == tbg

    def cut_fg(_, carry):
        lo, hi = carry
        mid = (lo + hi) // 2
        ok = jnp.sum((tie_fg & (idx < mid)).astype(jnp.int32)) >= m_fg
        return jnp.where(ok, lo, mid + 1), jnp.where(ok, mid, hi)

    _, cfg = jax.lax.fori_loop(
        0, 15, cut_fg, (jnp.int32(0), jnp.int32(NPIX + 1)))

    def cut_bg(_, carry):
        lo, hi = carry
        mid = (lo + hi) // 2
        ok = jnp.sum((tie_bg & (idx < mid)).astype(jnp.int32)) >= m_bg
        return jnp.where(ok, lo, mid + 1), jnp.where(ok, mid, hi)

    _, cbg = jax.lax.fori_loop(
        0, 15, cut_bg, (jnp.int32(0), jnp.int32(NPIX + 1)))

    fgf = ((mbits > tfg) | (tie_fg & (idx < cfg))).astype(jnp.float32)
    bgf = ((mbits < tbg) | (tie_bg & (idx < cbg))).astype(jnp.float32)

    lane = jax.lax.broadcasted_iota(jnp.int32, (1, LANES), 1).astype(jnp.float32)

    def hist_body(i, carry):
        hfg, hbg = carry
        fi = i.astype(jnp.float32)
        eq = (cmap == fi).astype(jnp.float32)
        sel = (lane == fi).astype(jnp.float32)
        return (hfg + jnp.sum(eq * fgf) * sel,
                hbg + jnp.sum(eq * bgf) * sel)

    zeros_l = jnp.zeros((1, LANES), jnp.float32)
    hfg, hbg = jax.lax.fori_loop(1, 125, hist_body, (zeros_l, zeros_l))

    valid = (lane < 125.0).astype(jnp.float32)
    hbg1 = hbg + valid
    dfg = hfg / (jnp.sum(hfg) + EPS)
    dbg = hbg1 / (jnp.sum(hbg1) + EPS)
    ratio = jnp.where(lane < 125.0, dfg / (dbg + dfg), 0.0)

    cmapc = jnp.minimum(cmap, 124.0)

    def gath_body(i, pr):
        fi = i.astype(jnp.float32)
        rv = jnp.sum(ratio * (lane == fi).astype(jnp.float32))
        return pr + (cmapc == fi).astype(jnp.float32) * rv

    pr = jax.lax.fori_loop(1, 125, gath_body,
                           jnp.zeros((H, W), jnp.float32))

    y2 = jnp.dot(jnp.dot(Mb, pr, preferred_element_type=jnp.float32, precision=jax.lax.Precision.HIGHEST), Mb.T,
                 preferred_element_type=jnp.float32, precision=jax.lax.Precision.HIGHEST)
    y2 = y2 - jnp.min(y2)
    y2 = y2 / (jnp.max(y2) + EPS)
    out_ref[0] = y2


def _compose_kernel(cur_ref, prev_ref, m_ref, out_ref):
    m = m_ref[0][None]
    out_ref[0] = prev_ref[0] * (1.0 - m) + cur_ref[0] * m


def kernel(video_clips):
    vf = video_clips.reshape(B, C * T, H, W)
    Mc = jnp.asarray(_BLUR_M)

    kv = jnp.arange(15, dtype=jnp.float32) - 7.0
    g1 = jnp.exp(-0.5 * (kv / 5.0) ** 2)
    g1 = g1 / g1.sum()
    k2bf = jnp.outer(g1, g1).astype(jnp.bfloat16).astype(jnp.float32)
    Amat = jnp.einsum('aij,ab->bij', jnp.asarray(_BAND_E), k2bf,
                      precision=jax.lax.Precision.HIGHEST)
    wv = jnp.arange(H, dtype=jnp.float32) - (H - 1) / 2.0
    gw = jnp.exp(-0.5 * (wv / (H / 3.0)) ** 2)
    gw = gw / gw.sum()
    win2 = jnp.outer(gw, gw)
    win = win2 / jnp.max(win2)
    Pmat = jnp.asarray(_PAD_P)

    diff, rgbmean = pl.pallas_call(
        _stats_kernel,
        grid=(B,),
        in_specs=[pl.BlockSpec((1, C * T, H, W), lambda b: (b, 0, 0, 0))],
        out_specs=[pl.BlockSpec((1, H, W), lambda b: (b, 0, 0)),
                   pl.BlockSpec((1, C, H, W), lambda b: (b, 0, 0, 0))],
        out_shape=[jax.ShapeDtypeStruct((B, H, W), jnp.float32),
                   jax.ShapeDtypeStruct((B, C, H, W), jnp.float32)],
    )(vf)

    mask2 = pl.pallas_call(
        _mid_kernel,
        grid=(B,),
        in_specs=[pl.BlockSpec((1, H, W), lambda b: (b, 0, 0)),
                  pl.BlockSpec((1, C, H, W), lambda b: (b, 0, 0, 0)),
                  pl.BlockSpec((15, H, H + 14), lambda b: (0, 0, 0)),
                  pl.BlockSpec((H + 14, H), lambda b: (0, 0)),
                  pl.BlockSpec((H, W), lambda b: (0, 0)),
                  pl.BlockSpec((H, W), lambda b: (0, 0))],
        out_specs=pl.BlockSpec((1, H, W), lambda b: (b, 0, 0)),
        out_shape=jax.ShapeDtypeStruct((B, H, W), jnp.float32),
    )(diff, rgbmean, Amat, Pmat, Mc, win)

    out = pl.pallas_call(
        _compose_kernel,
        grid=(B,),
        in_specs=[pl.BlockSpec((1, C * T, H, W), lambda b: (b, 0, 0, 0)),
                  pl.BlockSpec((1, C * T, H, W),
                               lambda b: ((b + B - 1) % B, 0, 0, 0)),
                  pl.BlockSpec((1, H, W), lambda b: (b, 0, 0))],
        out_specs=pl.BlockSpec((1, C * T, H, W), lambda b: (b, 0, 0, 0)),
        out_shape=jax.ShapeDtypeStruct((B, C * T, H, W), jnp.float32),
    )(vf, vf, mask2)

    return out.reshape(B, C, T, H, W)


# speedup vs baseline: 21.2009x; 2.6683x over previous
import numpy as np
import jax
import jax.numpy as jnp
from jax.experimental import pallas as pl
from jax.experimental.pallas import tpu as pltpu

EPS = 1e-8
B, C, T, H, W = 16, 3, 32, 112, 112
NPIX = H * W
TOPK = int(0.1 * H * W)
LANES = 128
ONE_BITS = 0x3F800000


def _gauss1d_np(ks, sigma):
    x = np.arange(ks, dtype=np.float32) - (ks - 1) / 2.0
    g = np.exp(-0.5 * (x / sigma) ** 2).astype(np.float32)
    return g / g.sum()


def _blur_matrix():
    k1 = _gauss1d_np(15, 5.0).astype(np.float64)
    t = np.arange(-7, H + 7)
    r = np.where(t < 0, -t, np.where(t > H - 1, 2 * (H - 1) - t, t))
    M = np.zeros((H, H), np.float64)
    for a in range(15):
        for i in range(H):
            M[i, r[i + a]] += k1[a]
    return M.astype(np.float32)


def _window_np():
    ky = _gauss1d_np(H, H / 3.0)
    kx = _gauss1d_np(W, W / 3.0)
    k = np.outer(ky, kx)
    return (k / k.max()).astype(np.float32)


_BLUR_M = _blur_matrix()
_WIN = _window_np()


def _reflect_pad_matrix():
    t = np.arange(-7, H + 7)
    r = np.where(t < 0, -t, np.where(t > H - 1, 2 * (H - 1) - t, t))
    P = np.zeros((H + 14, H), np.float32)
    P[np.arange(H + 14), r] = 1.0
    return P


def _band_placement():
    E = np.zeros((15, H, H + 14), np.float32)
    for a in range(15):
        E[a, np.arange(H), np.arange(H) + a] = 1.0
    return E


_PAD_P = _reflect_pad_matrix()
_BAND_E = _band_placement()


def _stats_kernel(v_ref, diff_ref, mean_ref):
    v = v_ref[0]
    acc = None
    for c in range(C):
        xc = v[c * T:(c + 1) * T]
        d = jnp.abs(xc[:-1] - xc[1:])
        acc = d if acc is None else acc + d
    diff_ref[0] = acc.mean(axis=0)
    for c in range(C):
        mean_ref[0, c] = v[c * T + 1:(c + 1) * T].mean(axis=0)


def _dot(a, b):
    return jnp.dot(a, b, preferred_element_type=jnp.float32,
                   precision=jax.lax.Precision.HIGHEST)


def _mid_kernel(diff_ref, rgb_ref, A_ref, P_ref, M_ref, win_ref, out_ref):
    Mb = M_ref[...]

    xbf = diff_ref[0].astype(jnp.bfloat16).astype(jnp.float32)
    P = P_ref[...]
    xp = _dot(_dot(P, xbf), P.T)
    acc = jnp.zeros((H, W), jnp.float32)
    for b in range(15):
        Tb = _dot(A_ref[b], xp)
        acc = acc + Tb[:, b:b + W]
    y = acc
    y = y - jnp.min(y)
    y = y / (jnp.max(y) + EPS)
    mask = y * win_ref[...]

    r = rgb_ref[0, 0]
    g = rgb_ref[0, 1]
    bl = rgb_ref[0, 2]
    maxc = jnp.maximum(jnp.maximum(r, g), bl)
    minc = jnp.minimum(jnp.minimum(r, g), bl)
    vch = maxc
    deltac = maxc - minc
    s = deltac / (maxc + EPS)
    dsafe = jnp.where(deltac == 0.0, 1.0, deltac)
    rc = (maxc - r) / dsafe
    gc = (maxc - g) / dsafe
    bc = (maxc - bl) / dsafe
    h = jnp.where(maxc == r, bc - gc,
                  jnp.where(maxc == g, 2.0 + rc - bc, 4.0 + gc - rc))
    h = (h / 6.0) % 1.0
    h = jnp.where(deltac == 0.0, 0.0, h)

    hx = (s * jnp.cos(h * (2 * np.pi)) + 1.0) / 2.0
    hy = (s * jnp.sin(h * (2 * np.pi)) + 1.0) / 2.0
    hq = jnp.round(hx * 4.0 + 1.0)
    sq = jnp.round(hy * 4.0 + 1.0)
    vq = jnp.round(vch * 4.0 + 1.0)
    cmap = hq + (sq - 1.0) * 5.0 + (vq - 1.0) * 25.0

    mbits = jax.lax.bitcast_convert_type(mask, jnp.int32)
    idx = (jax.lax.broadcasted_iota(jnp.int32, (H, W), 0) * W
           + jax.lax.broadcasted_iota(jnp.int32, (H, W), 1))

    def thr_body(_, carry):
        flo, fhi, blo, bhi = carry
        fmid = flo + (fhi - flo) // 2
        bmid = (blo + bhi) // 2
        fok = jnp.sum((mbits >= fmid).astype(jnp.int32)) >= TOPK
        bok = jnp.sum((mbits <= bmid).astype(jnp.int32)) >= TOPK
        return (jnp.where(fok, fmid, flo), jnp.where(fok, fhi, fmid),
                jnp.where(bok, blo, bmid + 1), jnp.where(bok, bmid, bhi))

    tfg, _, _, tbg = jax.lax.fori_loop(
        0, 31, thr_body,
        (jnp.int32(0), jnp.int32(ONE_BITS + 1),
         jnp.int32(0), jnp.int32(ONE_BITS)))

    m_fg = TOPK - jnp.sum((mbits > tfg).astype(jnp.int32))
    tie_fg = mbits == tfg
    m_bg = TOPK - jnp.sum((mbits < tbg).astype(jnp.int32))
    tie_bg = mbits == tbg

    def cut_body(_, carry):
        flo, fhi, blo, bhi = carry
        fmid = (flo + fhi) // 2
        bmid = (blo + bhi) // 2
        fok = jnp.sum((tie_fg & (idx < fmid)).astype(jnp.int32)) >= m_fg
        bok = jnp.sum((tie_bg & (idx < bmid)).astype(jnp.int32)) >= m_bg
        return (jnp.where(fok, flo, fmid + 1), jnp.where(fok, fmid, fhi),
                jnp.where(bok, blo, bmid + 1), jnp.where(bok, bmid, bhi))

    _, cfg, _, cbg = jax.lax.fori_loop(
        0, 15, cut_body,
        (jnp.int32(0), jnp.int32(NPIX + 1),
         jnp.int32(0), jnp.int32(NPIX + 1)))

    fgf = ((mbits > tfg) | (tie_fg & (idx < cfg))).astype(jnp.float32)
    bgf = ((mbits < tbg) | (tie_bg & (idx < cbg))).astype(jnp.float32)

    sub8 = jax.lax.broadcasted_iota(jnp.int32, (8, 1, 1), 0).astype(jnp.float32)
    lane16 = jax.lax.broadcasted_iota(jnp.int32, (8, 16), 1).astype(jnp.float32)
    sub16 = jax.lax.broadcasted_iota(jnp.int32, (8, 16), 0).astype(jnp.float32)
    binidx = 8.0 * lane16 + sub16
    fgf3 = fgf[None]
    bgf3 = bgf[None]
    hfg = jnp.zeros((8, 16), jnp.float32)
    hbg = jnp.zeros((8, 16), jnp.float32)
    for g in range(16):
        binvals = sub8 + (8.0 * g)
        eq = (cmap[None] == binvals).astype(jnp.float32)
        cf = jnp.sum(eq * fgf3, axis=(1, 2)).reshape(8, 1)
        cb = jnp.sum(eq * bgf3, axis=(1, 2)).reshape(8, 1)
        sel = (lane16 == float(g)).astype(jnp.float32)
        hfg = hfg + cf * sel
        hbg = hbg + cb * sel

    valid = (binidx < 125.0).astype(jnp.float32)
    nfg = hfg * valid
    nbg = hbg * valid + valid
    dfg = nfg / (jnp.sum(nfg) + EPS)
    dbg = nbg / (jnp.sum(nbg) + EPS)
    ratio = jnp.where(binidx < 125.0, dfg / (dbg + dfg), 0.0)

    cmapc = jnp.minimum(cmap, 124.0)
    pr = jnp.zeros((H, W), jnp.float32)
    for g in range(16):
        binvals = sub8 + (8.0 * g)
        eq = (cmapc[None] == binvals).astype(jnp.float32)
        rcol = ratio[:, g].reshape(8, 1, 1)
        pr = pr + jnp.sum(eq * rcol, axis=0)

    y2 = jnp.dot(jnp.dot(Mb, pr, preferred_element_type=jnp.float32, precision=jax.lax.Precision.HIGHEST), Mb.T,
                 preferred_element_type=jnp.float32, precision=jax.lax.Precision.HIGHEST)
    y2 = y2 - jnp.min(y2)
    y2 = y2 / (jnp.max(y2) + EPS)
    out_ref[0] = y2


def _compose_kernel(cur_ref, prev_ref, m_ref, out_ref):
    m = m_ref[0][None]
    out_ref[0] = prev_ref[0] * (1.0 - m) + cur_ref[0] * m


def kernel(video_clips):
    vf = video_clips.reshape(B, C * T, H, W)
    Mc = jnp.asarray(_BLUR_M)

    kv = jnp.arange(15, dtype=jnp.float32) - 7.0
    g1 = jnp.exp(-0.5 * (kv / 5.0) ** 2)
    g1 = g1 / g1.sum()
    k2bf = jnp.outer(g1, g1).astype(jnp.bfloat16).astype(jnp.float32)
    Amat = jnp.einsum('aij,ab->bij', jnp.asarray(_BAND_E), k2bf,
                      precision=jax.lax.Precision.HIGHEST)
    wv = jnp.arange(H, dtype=jnp.float32) - (H - 1) / 2.0
    gw = jnp.exp(-0.5 * (wv / (H / 3.0)) ** 2)
    gw = gw / gw.sum()
    win2 = jnp.outer(gw, gw)
    win = win2 / jnp.max(win2)
    Pmat = jnp.asarray(_PAD_P)

    diff, rgbmean = pl.pallas_call(
        _stats_kernel,
        grid=(B,),
        in_specs=[pl.BlockSpec((1, C * T, H, W), lambda b: (b, 0, 0, 0))],
        out_specs=[pl.BlockSpec((1, H, W), lambda b: (b, 0, 0)),
                   pl.BlockSpec((1, C, H, W), lambda b: (b, 0, 0, 0))],
        out_shape=[jax.ShapeDtypeStruct((B, H, W), jnp.float32),
                   jax.ShapeDtypeStruct((B, C, H, W), jnp.float32)],
        compiler_params=pltpu.CompilerParams(
            dimension_semantics=("parallel",)),
    )(vf)

    mask2 = pl.pallas_call(
        _mid_kernel,
        grid=(B,),
        in_specs=[pl.BlockSpec((1, H, W), lambda b: (b, 0, 0)),
                  pl.BlockSpec((1, C, H, W), lambda b: (b, 0, 0, 0)),
                  pl.BlockSpec((15, H, H + 14), lambda b: (0, 0, 0)),
                  pl.BlockSpec((H + 14, H), lambda b: (0, 0)),
                  pl.BlockSpec((H, W), lambda b: (0, 0)),
                  pl.BlockSpec((H, W), lambda b: (0, 0))],
        out_specs=pl.BlockSpec((1, H, W), lambda b: (b, 0, 0)),
        out_shape=jax.ShapeDtypeStruct((B, H, W), jnp.float32),
        compiler_params=pltpu.CompilerParams(
            dimension_semantics=("parallel",)),
    )(diff, rgbmean, Amat, Pmat, Mc, win)

    out = pl.pallas_call(
        _compose_kernel,
        grid=(B,),
        in_specs=[pl.BlockSpec((1, C * T, H, W), lambda b: (b, 0, 0, 0)),
                  pl.BlockSpec((1, C * T, H, W),
                               lambda b: ((b + B - 1) % B, 0, 0, 0)),
                  pl.BlockSpec((1, H, W), lambda b: (b, 0, 0))],
        out_specs=pl.BlockSpec((1, C * T, H, W), lambda b: (b, 0, 0, 0)),
        out_shape=jax.ShapeDtypeStruct((B, C * T, H, W), jnp.float32),
        compiler_params=pltpu.CompilerParams(
            dimension_semantics=("parallel",)),
    )(vf, vf, mask2)

    return out.reshape(B, C, T, H, W)
